# baseline (speedup 1.0000x reference)
import functools

import jax
import jax.numpy as jnp
import numpy as np
from jax.experimental import pallas as pl
from jax.experimental.pallas import tpu as pltpu

_G = 16
_TR = 1024
_W = 56
_LIN = _G * _W
_LOUT = (_G // 2) * (_W // 2)


@functools.lru_cache(maxsize=None)
def _pool_matrix_np(w: int, g: int):
    lin = g * w
    wo = w // 2
    lout = (g // 2) * wo
    lane = np.arange(lin)
    dst = (lane // (2 * w)) * wo + (lane % w) // 2
    m = np.zeros((lin, lout), dtype=np.float32)
    m[lane, dst] = 0.25
    return m


def _pool_body(x_ref, r_ref, o_ref):
    o_ref[...] = jnp.dot(
        x_ref[...], r_ref[...], preferred_element_type=jnp.float32
    )


def kernel(x):
    n, c, h, w = x.shape
    assert (h % 2, w % 2) == (0, 0)
    rows = n * c * h // _G
    xs = x.reshape(rows, _LIN)
    r_mat = jnp.asarray(_pool_matrix_np(w, _G))
    tr = _TR if rows % _TR == 0 else rows
    grid = (rows // tr,)
    out = pl.pallas_call(
        _pool_body,
        out_shape=jax.ShapeDtypeStruct((rows, _LOUT), jnp.float32),
        grid=grid,
        in_specs=[
            pl.BlockSpec((tr, _LIN), lambda i: (i, 0)),
            pl.BlockSpec((_LIN, _LOUT), lambda i: (0, 0)),
        ],
        out_specs=pl.BlockSpec((tr, _LOUT), lambda i: (i, 0)),
        compiler_params=pltpu.CompilerParams(
            dimension_semantics=("parallel",),
            vmem_limit_bytes=64 * 1024 * 1024,
        ),
        cost_estimate=pl.CostEstimate(
            flops=2 * rows * _LIN * _LOUT,
            transcendentals=0,
            bytes_accessed=(rows * (_LIN + _LOUT) + _LIN * _LOUT) * 4,
        ),
    )(xs, r_mat)
    return out.reshape(n, c, h // 2, w // 2)


# speedup vs baseline: 1.5861x; 1.5861x over previous
import functools

import jax
import jax.numpy as jnp
import numpy as np
from jax.experimental import pallas as pl
from jax.experimental.pallas import tpu as pltpu

_W = 56
_WO = _W // 2
_B = 64


@functools.lru_cache(maxsize=None)
def _hpool_matrix_np(w: int):
    m = np.zeros((w, w // 2), dtype=np.float32)
    m[np.arange(w), np.arange(w) // 2] = 0.25
    return m


def _pool_body(x_ref, r_ref, o_ref):
    m = x_ref.shape[0] // 2
    v = (x_ref[pl.ds(0, m, stride=2), :]
         + x_ref[pl.ds(1, m, stride=2), :])
    h = jnp.dot(v, r_ref[...], preferred_element_type=jnp.float32)
    o_ref[...] = h.reshape(o_ref.shape)


def kernel(x):
    n, c, h, w = x.shape
    imgs = n * c
    xs = x.reshape(imgs * h, w)
    r_mat = jnp.asarray(_hpool_matrix_np(w))
    b = _B if imgs % _B == 0 else imgs
    out = pl.pallas_call(
        _pool_body,
        out_shape=jax.ShapeDtypeStruct((imgs, h // 2, w // 2), jnp.float32),
        grid=(imgs // b,),
        in_specs=[
            pl.BlockSpec((h * b, w), lambda i: (i, 0)),
            pl.BlockSpec((w, w // 2), lambda i: (0, 0)),
        ],
        out_specs=pl.BlockSpec((b, h // 2, w // 2), lambda i: (i, 0, 0)),
        compiler_params=pltpu.CompilerParams(
            dimension_semantics=("parallel",),
            vmem_limit_bytes=64 * 1024 * 1024,
        ),
        cost_estimate=pl.CostEstimate(
            flops=2 * imgs * h * w // 2 * (w // 2),
            transcendentals=0,
            bytes_accessed=(imgs * h * w + imgs * (h // 2) * (w // 2)) * 4,
        ),
    )(xs, r_mat)
    return out.reshape(n, c, h // 2, w // 2)


# speedup vs baseline: 7.3468x; 4.6320x over previous
import jax
import jax.numpy as jnp
from jax.experimental import pallas as pl
from jax.experimental.pallas import tpu as pltpu


def _pool_body(x_ref, o_ref, scr_ref):
    v = (x_ref[:, 0, :, :] + x_ref[:, 1, :, :]) * 0.25
    scr_ref[...] = v
    for j in range(o_ref.shape[1]):
        o_ref[0, j, :, :] = scr_ref[:, 2 * j, :] + scr_ref[:, 2 * j + 1, :]


def kernel(x):
    n, c, h, w = x.shape
    ho, wo = h // 2, w // 2
    xt = jnp.transpose(x, (0, 2, 3, 1))
    out = pl.pallas_call(
        _pool_body,
        out_shape=jax.ShapeDtypeStruct((ho, wo, n, c), jnp.float32),
        grid=(ho,),
        in_specs=[
            pl.BlockSpec((n, 2, w, c), lambda i: (0, i, 0, 0)),
        ],
        out_specs=pl.BlockSpec((1, wo, n, c), lambda i: (i, 0, 0, 0)),
        scratch_shapes=[pltpu.VMEM((n, w, c), jnp.float32)],
        compiler_params=pltpu.CompilerParams(
            dimension_semantics=("parallel",),
            vmem_limit_bytes=64 * 1024 * 1024,
        ),
        cost_estimate=pl.CostEstimate(
            flops=3 * n * c * ho * wo,
            transcendentals=0,
            bytes_accessed=(n * c * h * w + n * c * ho * wo) * 4,
        ),
    )(xt)
    return jnp.transpose(out, (2, 3, 0, 1))


# speedup vs baseline: 7.7559x; 1.0557x over previous
import jax
import jax.numpy as jnp
from jax.experimental import pallas as pl
from jax.experimental.pallas import tpu as pltpu


def _pool_body(x_ref, o_ref, scr_ref):
    wo = o_ref.shape[1]
    s = (x_ref[:, 0, pl.ds(0, wo, stride=2), :]
         + x_ref[:, 0, pl.ds(1, wo, stride=2), :]
         + x_ref[:, 1, pl.ds(0, wo, stride=2), :]
         + x_ref[:, 1, pl.ds(1, wo, stride=2), :]) * 0.25
    scr_ref[...] = s
    for j in range(wo):
        o_ref[0, j, :, :] = scr_ref[:, j, :]


def kernel(x):
    n, c, h, w = x.shape
    ho, wo = h // 2, w // 2
    xt = jnp.transpose(x, (0, 2, 3, 1))
    out = pl.pallas_call(
        _pool_body,
        out_shape=jax.ShapeDtypeStruct((ho, wo, n, c), jnp.float32),
        grid=(ho,),
        in_specs=[
            pl.BlockSpec((n, 2, w, c), lambda i: (0, i, 0, 0)),
        ],
        out_specs=pl.BlockSpec((1, wo, n, c), lambda i: (i, 0, 0, 0)),
        scratch_shapes=[pltpu.VMEM((n, wo, c), jnp.float32)],
        compiler_params=pltpu.CompilerParams(
            dimension_semantics=("parallel",),
            vmem_limit_bytes=64 * 1024 * 1024,
        ),
        cost_estimate=pl.CostEstimate(
            flops=4 * n * c * ho * wo,
            transcendentals=0,
            bytes_accessed=(n * c * h * w + n * c * ho * wo) * 4,
        ),
    )(xt)
    return jnp.transpose(out, (2, 3, 0, 1))


# speedup vs baseline: 9.9124x; 1.2780x over previous
import jax
import jax.numpy as jnp
from jax.experimental import pallas as pl
from jax.experimental.pallas import tpu as pltpu

_BI = 2


def _pool_body(x_ref, o_ref, scr_ref):
    bi, wo = o_ref.shape[0], o_ref.shape[1]
    for ii in range(bi):
        s = (x_ref[:, 2 * ii, pl.ds(0, wo, stride=2), :]
             + x_ref[:, 2 * ii, pl.ds(1, wo, stride=2), :]
             + x_ref[:, 2 * ii + 1, pl.ds(0, wo, stride=2), :]
             + x_ref[:, 2 * ii + 1, pl.ds(1, wo, stride=2), :]) * 0.25
        scr_ref[ii, :, :, :] = s
    for ii in range(bi):
        for j in range(wo):
            o_ref[ii, j, :, :] = scr_ref[ii, :, j, :]


def kernel(x):
    n, c, h, w = x.shape
    ho, wo = h // 2, w // 2
    xt = jnp.transpose(x, (0, 2, 3, 1))
    out = pl.pallas_call(
        _pool_body,
        out_shape=jax.ShapeDtypeStruct((ho, wo, n, c), jnp.float32),
        grid=(ho // _BI,),
        in_specs=[
            pl.BlockSpec((n, 2 * _BI, w, c), lambda i: (0, i, 0, 0)),
        ],
        out_specs=pl.BlockSpec((_BI, wo, n, c), lambda i: (i, 0, 0, 0)),
        scratch_shapes=[pltpu.VMEM((_BI, n, wo, c), jnp.float32)],
        compiler_params=pltpu.CompilerParams(
            dimension_semantics=("parallel",),
            vmem_limit_bytes=64 * 1024 * 1024,
        ),
        cost_estimate=pl.CostEstimate(
            flops=4 * n * c * ho * wo,
            transcendentals=0,
            bytes_accessed=(n * c * h * w + n * c * ho * wo) * 4,
        ),
    )(xt)
    return jnp.transpose(out, (2, 3, 0, 1))


# speedup vs baseline: 10.6963x; 1.0791x over previous
import jax
import jax.numpy as jnp
from jax.experimental import pallas as pl
from jax.experimental.pallas import tpu as pltpu

_BI = 4


def _pool_body(x_ref, o_ref, scr_ref):
    bi, wo = o_ref.shape[0], o_ref.shape[1]
    for ii in range(bi):
        s = (x_ref[:, 2 * ii, pl.ds(0, wo, stride=2), :]
             + x_ref[:, 2 * ii, pl.ds(1, wo, stride=2), :]
             + x_ref[:, 2 * ii + 1, pl.ds(0, wo, stride=2), :]
             + x_ref[:, 2 * ii + 1, pl.ds(1, wo, stride=2), :]) * 0.25
        scr_ref[ii, :, :, :] = s
    for ii in range(bi):
        for j in range(wo):
            o_ref[ii, j, :, :] = scr_ref[ii, :, j, :]


def kernel(x):
    n, c, h, w = x.shape
    ho, wo = h // 2, w // 2
    xt = jnp.transpose(x, (0, 2, 3, 1))
    out = pl.pallas_call(
        _pool_body,
        out_shape=jax.ShapeDtypeStruct((ho, wo, n, c), jnp.float32),
        grid=(ho // _BI,),
        in_specs=[
            pl.BlockSpec((n, 2 * _BI, w, c), lambda i: (0, i, 0, 0)),
        ],
        out_specs=pl.BlockSpec((_BI, wo, n, c), lambda i: (i, 0, 0, 0)),
        scratch_shapes=[pltpu.VMEM((_BI, n, wo, c), jnp.float32)],
        compiler_params=pltpu.CompilerParams(
            dimension_semantics=("parallel",),
            vmem_limit_bytes=64 * 1024 * 1024,
        ),
        cost_estimate=pl.CostEstimate(
            flops=4 * n * c * ho * wo,
            transcendentals=0,
            bytes_accessed=(n * c * h * w + n * c * ho * wo) * 4,
        ),
    )(xt)
    return jnp.transpose(out, (2, 3, 0, 1))


# speedup vs baseline: 10.9532x; 1.0240x over previous
import jax
import jax.numpy as jnp
from jax.experimental import pallas as pl
from jax.experimental.pallas import tpu as pltpu

_BI = 7


def _pool_body(x_ref, o_ref, scr_ref):
    bi, wo = o_ref.shape[0], o_ref.shape[1]
    for ii in range(bi):
        s = (x_ref[:, 2 * ii, pl.ds(0, wo, stride=2), :]
             + x_ref[:, 2 * ii, pl.ds(1, wo, stride=2), :]
             + x_ref[:, 2 * ii + 1, pl.ds(0, wo, stride=2), :]
             + x_ref[:, 2 * ii + 1, pl.ds(1, wo, stride=2), :]) * 0.25
        scr_ref[ii, :, :, :] = s
    for ii in range(bi):
        for j in range(wo):
            o_ref[ii, j, :, :] = scr_ref[ii, :, j, :]


def kernel(x):
    n, c, h, w = x.shape
    ho, wo = h // 2, w // 2
    xt = jnp.transpose(x, (0, 2, 3, 1))
    out = pl.pallas_call(
        _pool_body,
        out_shape=jax.ShapeDtypeStruct((ho, wo, n, c), jnp.float32),
        grid=(ho // _BI,),
        in_specs=[
            pl.BlockSpec((n, 2 * _BI, w, c), lambda i: (0, i, 0, 0)),
        ],
        out_specs=pl.BlockSpec((_BI, wo, n, c), lambda i: (i, 0, 0, 0)),
        scratch_shapes=[pltpu.VMEM((_BI, n, wo, c), jnp.float32)],
        compiler_params=pltpu.CompilerParams(
            dimension_semantics=("parallel",),
            vmem_limit_bytes=64 * 1024 * 1024,
        ),
        cost_estimate=pl.CostEstimate(
            flops=4 * n * c * ho * wo,
            transcendentals=0,
            bytes_accessed=(n * c * h * w + n * c * ho * wo) * 4,
        ),
    )(xt)
    return jnp.transpose(out, (2, 3, 0, 1))


# speedup vs baseline: 11.0274x; 1.0068x over previous
import jax
import jax.numpy as jnp
from jax.experimental import pallas as pl
from jax.experimental.pallas import tpu as pltpu

_BI = 7


def _pool_body(x1_ref, x2_ref, o_ref, scr_ref):
    bi, wo = o_ref.shape[0], o_ref.shape[1]
    hn = x1_ref.shape[0]
    for ii in range(bi):
        for k, xr in enumerate((x1_ref, x2_ref)):
            s = (xr[:, 2 * ii, pl.ds(0, wo, stride=2), :]
                 + xr[:, 2 * ii, pl.ds(1, wo, stride=2), :]
                 + xr[:, 2 * ii + 1, pl.ds(0, wo, stride=2), :]
                 + xr[:, 2 * ii + 1, pl.ds(1, wo, stride=2), :]) * 0.25
            scr_ref[ii, k * hn:(k + 1) * hn, :, :] = s
    for ii in range(bi):
        for j in range(wo):
            o_ref[ii, j, :, :] = scr_ref[ii, :, j, :]


def kernel(x):
    n, c, h, w = x.shape
    ho, wo = h // 2, w // 2
    xt = jnp.transpose(x, (0, 2, 3, 1))
    out = pl.pallas_call(
        _pool_body,
        out_shape=jax.ShapeDtypeStruct((ho, wo, n, c), jnp.float32),
        grid=(ho // _BI,),
        in_specs=[
            pl.BlockSpec((n // 2, 2 * _BI, w, c), lambda i: (0, i, 0, 0)),
            pl.BlockSpec((n // 2, 2 * _BI, w, c), lambda i: (1, i, 0, 0)),
        ],
        out_specs=pl.BlockSpec((_BI, wo, n, c), lambda i: (i, 0, 0, 0)),
        scratch_shapes=[pltpu.VMEM((_BI, n, wo, c), jnp.float32)],
        compiler_params=pltpu.CompilerParams(
            dimension_semantics=("parallel",),
            vmem_limit_bytes=64 * 1024 * 1024,
        ),
        cost_estimate=pl.CostEstimate(
            flops=4 * n * c * ho * wo,
            transcendentals=0,
            bytes_accessed=(n * c * h * w + n * c * ho * wo) * 4,
        ),
    )(xt, xt)
    return jnp.transpose(out, (2, 3, 0, 1))


# speedup vs baseline: 11.0326x; 1.0005x over previous
import jax
import jax.numpy as jnp
from jax.experimental import pallas as pl
from jax.experimental.pallas import tpu as pltpu

_BI = 7


def _pool_body(x_ref, o_ref, scr_ref):
    bi, wo = o_ref.shape[0], o_ref.shape[1]
    for ii in range(bi):
        s = (x_ref[:, 2 * ii, pl.ds(0, wo, stride=2), :]
             + x_ref[:, 2 * ii, pl.ds(1, wo, stride=2), :]
             + x_ref[:, 2 * ii + 1, pl.ds(0, wo, stride=2), :]
             + x_ref[:, 2 * ii + 1, pl.ds(1, wo, stride=2), :]) * 0.25
        scr_ref[ii, :, :, :] = s
    for ii in range(bi):
        for j in range(wo):
            o_ref[ii, j, :, :] = scr_ref[ii, :, j, :]


def kernel(x):
    n, c, h, w = x.shape
    ho, wo = h // 2, w // 2
    xt = jnp.transpose(x, (0, 2, 3, 1))
    out = pl.pallas_call(
        _pool_body,
        out_shape=jax.ShapeDtypeStruct((ho, wo, n, c), jnp.float32),
        grid=(ho // _BI,),
        in_specs=[
            pl.BlockSpec((n, 2 * _BI, w, c), lambda i: (0, i, 0, 0)),
        ],
        out_specs=pl.BlockSpec((_BI, wo, n, c), lambda i: (i, 0, 0, 0)),
        scratch_shapes=[pltpu.VMEM((_BI, n, wo, c), jnp.float32)],
        compiler_params=pltpu.CompilerParams(
            dimension_semantics=("parallel",),
            vmem_limit_bytes=64 * 1024 * 1024,
        ),
        cost_estimate=pl.CostEstimate(
            flops=4 * n * c * ho * wo,
            transcendentals=0,
            bytes_accessed=(n * c * h * w + n * c * ho * wo) * 4,
        ),
    )(xt)
    return jnp.transpose(out, (2, 3, 0, 1))
